# Initial kernel scaffold; baseline (speedup 1.0000x reference)
import jax
import jax.numpy as jnp
from jax.experimental import pallas as pl


def kernel(x, edge_index, W_self, b_self, W_disc, b_disc, W_att1, b_att1, W_att2):
    raise NotImplementedError("write your pallas kernel here")


# speedup vs baseline: 5.4500x; 5.4500x over previous
import functools

import jax
import jax.numpy as jnp
from jax import lax
from jax.experimental import pallas as pl
from jax.experimental.pallas import tpu as pltpu
from jax.experimental.pallas import tpu_sc as plsc

N = 10000
NP = 10240
E = 320000
D = 128
NC = 2
NS = 16
ROWS_PER_TILE = NP // NS
EDGES_PER_TILE = E // NS
CHUNK = 80
NCHUNK = EDGES_PER_TILE // CHUNK
DEG_ROWS = 128

BN = 1000


def _sc_segment_sums(x, eidx_flat, zeros_rows, iota128):
    mesh = plsc.VectorSubcoreMesh(
        core_axis_name="c", subcore_axis_name="s", num_cores=NC, num_subcores=NS
    )

    @functools.partial(
        pl.kernel,
        out_type=(
            jax.ShapeDtypeStruct((NC, NP, D), jnp.float32),
            jax.ShapeDtypeStruct((NC, DEG_ROWS, 128), jnp.float32),
        ),
        mesh=mesh,
        scratch_types=[
            pltpu.VMEM((CHUNK,), jnp.int32),
            pltpu.VMEM((CHUNK,), jnp.int32),
            pltpu.VMEM((CHUNK, D), jnp.float32),
            pltpu.VMEM((DEG_ROWS, 128), jnp.float32),
            pltpu.VMEM((128,), jnp.int32),
            pltpu.VMEM_SHARED((NP, D), jnp.float32),
            pltpu.VMEM_SHARED((DEG_ROWS, 128), jnp.float32),
            pltpu.SemaphoreType.DMA,
        ],
        compiler_params=pltpu.CompilerParams(needs_layout_passes=False),
    )
    def seg_kernel(x_hbm, eidx_hbm, zeros_hbm, iota_hbm,
                   s_out_hbm, deg_out_hbm,
                   gidx_v, sidx_v, rows_v, deg_v, iota_v, acc, acc_deg, sem):
        cid = lax.axis_index("c")
        sid = lax.axis_index("s")
        r0 = sid * ROWS_PER_TILE
        d0 = sid * (DEG_ROWS // NS)
        pltpu.sync_copy(zeros_hbm, acc.at[pl.ds(r0, ROWS_PER_TILE)])
        pltpu.sync_copy(zeros_hbm.at[pl.ds(0, DEG_ROWS)], deg_v)
        pltpu.sync_copy(zeros_hbm.at[pl.ds(0, DEG_ROWS // NS)],
                        acc_deg.at[pl.ds(d0, DEG_ROWS // NS)])
        pltpu.sync_copy(iota_hbm, iota_v)
        plsc.subcore_barrier()

        e0 = sid * EDGES_PER_TILE
        ones16 = jnp.ones((16,), jnp.float32)

        def body(k, carry):
            base = e0 + k * CHUNK
            pltpu.sync_copy(eidx_hbm.at[pl.ds(cid * E + base, CHUNK)], gidx_v)
            pltpu.sync_copy(eidx_hbm.at[pl.ds((1 - cid) * E + base, CHUNK)], sidx_v)
            pltpu.async_copy(x_hbm.at[gidx_v], rows_v, sem).wait()
            pltpu.sync_copy(rows_v, acc.at[sidx_v], add=True)
            for j in range(CHUNK // 16):
                idx = sidx_v[pl.ds(j * 16, 16)]
                plsc.addupdate_scatter(
                    deg_v,
                    [lax.shift_right_logical(idx, 7), lax.bitwise_and(idx, 127)],
                    ones16)
            return carry

        lax.fori_loop(0, NCHUNK, body, 0)
        plsc.subcore_barrier()
        pltpu.sync_copy(deg_v, acc_deg.at[iota_v], add=True)
        plsc.subcore_barrier()
        pltpu.sync_copy(
            acc.at[pl.ds(r0, ROWS_PER_TILE)],
            s_out_hbm.at[cid, pl.ds(r0, ROWS_PER_TILE)],
        )
        pltpu.sync_copy(
            acc_deg.at[pl.ds(d0, DEG_ROWS // NS)],
            deg_out_hbm.at[cid, pl.ds(d0, DEG_ROWS // NS)],
        )

    return seg_kernel(x, eidx_flat, zeros_rows, iota128)


def _combine_body(x_ref, sa_ref, so_ref, da_ref, do_ref,
                  wst_ref, wbt_ref, wdt_ref, wa1_ref, wa2_ref,
                  bs_ref, bd_ref, ba1_ref, out_ref):
    xb = x_ref[...]
    xs = jnp.dot(xb, wst_ref[...], preferred_element_type=jnp.float32) + bs_ref[...]
    xbB = jnp.dot(xb, wbt_ref[...], preferred_element_type=jnp.float32) + bd_ref[...]
    wdt = wdt_ref[...]
    inc = jnp.dot(sa_ref[0], wdt, preferred_element_type=jnp.float32) + da_ref[...] * xbB
    outg = jnp.dot(so_ref[0], wdt, preferred_element_type=jnp.float32) + do_ref[...] * xbB
    wa1 = wa1_ref[...]
    wa2 = wa2_ref[...]
    ba1 = ba1_ref[...]

    def logit(r):
        h = jnp.tanh(jnp.dot(r, wa1, preferred_element_type=jnp.float32) + ba1)
        return jnp.sum(h * wa2, axis=1, keepdims=True)

    l0, l1, l2 = logit(xs), logit(inc), logit(outg)
    m = jnp.maximum(jnp.maximum(l0, l1), l2)
    e0 = jnp.exp(l0 - m)
    e1 = jnp.exp(l1 - m)
    e2 = jnp.exp(l2 - m)
    out_ref[...] = (e0 * xs + e1 * inc + e2 * outg) / (e0 + e1 + e2)


def _tc_combine(x, S, din, dout, WsT, WbT, WdT, Wa1T, wa2, bs, bd, ba1):
    grid = (N // BN,)
    wspec = lambda shape: pl.BlockSpec(shape, lambda i: tuple(0 for _ in shape))
    return pl.pallas_call(
        _combine_body,
        grid=grid,
        in_specs=[
            pl.BlockSpec((BN, D), lambda i: (i, 0)),
            pl.BlockSpec((1, BN, D), lambda i: (0, i, 0)),
            pl.BlockSpec((1, BN, D), lambda i: (1, i, 0)),
            pl.BlockSpec((BN, 1), lambda i: (i, 0)),
            pl.BlockSpec((BN, 1), lambda i: (i, 0)),
            wspec((D, D)),
            wspec((D, D)),
            wspec((D, D)),
            wspec((D, 16)),
            wspec((1, 16)),
            wspec((1, D)),
            wspec((1, D)),
            wspec((1, 16)),
        ],
        out_specs=pl.BlockSpec((BN, D), lambda i: (i, 0)),
        out_shape=jax.ShapeDtypeStruct((N, D), jnp.float32),
    )(x, S, S, din, dout, WsT, WbT, WdT, Wa1T, wa2, bs, bd, ba1)


def kernel(x, edge_index, W_self, b_self, W_disc, b_disc, W_att1, b_att1, W_att2):
    x = x.astype(jnp.float32)
    eidx_flat = edge_index.astype(jnp.int32).reshape(2 * E)
    zeros_rows = jnp.zeros((ROWS_PER_TILE, D), jnp.float32)
    iota128 = jnp.arange(128, dtype=jnp.int32)
    S, degH = _sc_segment_sums(x, eidx_flat, zeros_rows, iota128)
    deg = degH.reshape(NC, DEG_ROWS * 128)[:, :N]
    din = deg[0].reshape(N, 1)
    dout = deg[1].reshape(N, 1)

    Wa = W_disc[:, :D]
    Wb = W_disc[:, D:]
    WsT = W_self.T
    WbT = Wb.T
    WdT = (Wa - Wb).T
    Wa1T = W_att1.T
    wa2 = W_att2.reshape(1, 16)
    bs = b_self.reshape(1, D)
    bd = b_disc.reshape(1, D)
    ba1 = b_att1.reshape(1, 16)
    return _tc_combine(x, S, din, dout, WsT, WbT, WdT, Wa1T, wa2, bs, bd, ba1)
